# baseline (speedup 1.0000x reference)
import functools

import jax
import jax.numpy as jnp
from jax import lax
from jax.experimental import pallas as pl
from jax.experimental.pallas import tpu as pltpu
from jax.experimental.pallas import tpu_sc as plsc

_NC = 2
_NS = 16
_L = 16

_ROWS = 32768
_C = 1024
_NW = _NC * _NS
_K = 8192
_ROWS_PER_W = _K // _NW
_CHUNK = 16
_G = _ROWS_PER_W // _CHUNK
_NCOL = _C // _L


def _sc_body(x_hbm, step_hbm, emb_hbm, out_hbm,
             inbuf, outbuf, idx_v, emb_v,
             in_sem0, in_sem1, out_sem0, out_sem1, gsem):
    in_sems = (in_sem0, in_sem1)
    out_sems = (out_sem0, out_sem1)
    wid = lax.axis_index("s") * _NC + lax.axis_index("c")
    base = wid * _ROWS_PER_W

    pltpu.sync_copy(step_hbm, idx_v)
    pltpu.async_copy(emb_hbm.at[idx_v], emb_v, gsem).wait()

    def start_fetch(g, b):
        pltpu.async_copy(x_hbm.at[pl.ds(base + g * _CHUNK, _CHUNK)],
                         inbuf.at[b], in_sems[b])

    def wait_fetch(b):
        pltpu.make_async_copy(x_hbm.at[pl.ds(0, _CHUNK)],
                              inbuf.at[b], in_sems[b]).wait()

    def start_wb(g, b):
        pltpu.async_copy(outbuf.at[b],
                         out_hbm.at[pl.ds(base + g * _CHUNK, _CHUNK)],
                         out_sems[b])

    def wait_wb(b):
        pltpu.make_async_copy(outbuf.at[b],
                              out_hbm.at[pl.ds(0, _CHUNK)],
                              out_sems[b]).wait()

    def compute(b):
        for j in range(_NCOL):
            e = emb_v[0, pl.ds(j * _L, _L)]

            def rbody(r, _, j=j, e=e):
                for rr in range(8):
                    row = r * 8 + rr
                    outbuf[b, row, pl.ds(j * _L, _L)] = (
                        inbuf[b, row, pl.ds(j * _L, _L)] + e)
                return 0

            lax.fori_loop(0, _CHUNK // 8, rbody, 0, unroll=False)

    start_fetch(0, 0)
    start_fetch(1, 1)

    def outer(i, _):
        g0 = i * 2
        for b in range(2):
            g = g0 + b
            wait_fetch(b)

            @pl.when(g >= 2)
            def _():
                wait_wb(b)

            compute(b)
            start_wb(g, b)

            @pl.when(g + 2 < _G)
            def _():
                start_fetch(g + 2, b)

        return 0

    lax.fori_loop(0, _G // 2, outer, 0, unroll=False)
    wait_wb(0)
    wait_wb(1)


def _sc_part(x2, step1, emb2):
    mesh = plsc.VectorSubcoreMesh(core_axis_name="c", subcore_axis_name="s",
                                  num_cores=_NC, num_subcores=_NS)
    run = functools.partial(
        pl.kernel,
        out_type=jax.ShapeDtypeStruct((_K, _C), jnp.float32),
        mesh=mesh,
        scratch_types=[
            pltpu.VMEM((2, _CHUNK, _C), jnp.float32),
            pltpu.VMEM((2, _CHUNK, _C), jnp.float32),
            pltpu.VMEM((1,), jnp.int32),
            pltpu.VMEM((1, _C), jnp.float32),
            pltpu.SemaphoreType.DMA,
            pltpu.SemaphoreType.DMA,
            pltpu.SemaphoreType.DMA,
            pltpu.SemaphoreType.DMA,
            pltpu.SemaphoreType.DMA,
        ],
    )(_sc_body)
    return run(x2, step1, emb2)


def kernel(x_layer, step, step_embedding):
    B, S, C = x_layer.shape
    x2 = x_layer.reshape(B * S, C)
    emb2 = step_embedding.reshape(step_embedding.shape[0], C)
    step1 = jnp.asarray(step, jnp.int32).reshape(1)

    sc_out = _sc_part(x2, step1, emb2)
    tc_out = x2[_K:] + emb2[step1[0]][None, :]
    out = jnp.concatenate([sc_out, tc_out], axis=0)
    return out.reshape(B, S, C)


# speedup vs baseline: 1.8469x; 1.8469x over previous
import functools

import jax
import jax.numpy as jnp
from jax import lax
from jax.experimental import pallas as pl
from jax.experimental.pallas import tpu as pltpu
from jax.experimental.pallas import tpu_sc as plsc

_NC = 2
_NS = 16

_C = 1024
_BLK = 2048


def _lookup_body(step_hbm, emb_hbm, out_hbm, step_smem, row_spmem):
    cid = lax.axis_index("c")

    @pl.when(cid == 0)
    def _():
        pltpu.sync_copy(step_hbm, step_smem)
        s = step_smem[0]
        pltpu.sync_copy(emb_hbm.at[s], row_spmem)
        pltpu.sync_copy(row_spmem, out_hbm)


def _sc_lookup(step1, emb2):
    mesh = plsc.ScalarSubcoreMesh(axis_name="c", num_cores=_NC)
    run = functools.partial(
        pl.kernel,
        out_type=jax.ShapeDtypeStruct((_C,), jnp.float32),
        mesh=mesh,
        scratch_types=[
            pltpu.SMEM((1,), jnp.int32),
            pltpu.VMEM_SHARED((_C,), jnp.float32),
        ],
    )(_lookup_body)
    return run(step1, emb2).reshape(1, _C)


def _tc_body(x_ref, row_ref, out_ref):
    out_ref[...] = x_ref[...] + row_ref[...]


def _tc_add(x2, row):
    n_rows = x2.shape[0]
    return pl.pallas_call(
        _tc_body,
        grid=(n_rows // _BLK,),
        in_specs=[
            pl.BlockSpec((_BLK, _C), lambda i: (i, 0)),
            pl.BlockSpec((1, _C), lambda i: (0, 0)),
        ],
        out_specs=pl.BlockSpec((_BLK, _C), lambda i: (i, 0)),
        out_shape=jax.ShapeDtypeStruct((n_rows, _C), jnp.float32),
    )(x2, row)


def kernel(x_layer, step, step_embedding):
    B, S, C = x_layer.shape
    x2 = x_layer.reshape(B * S, C)
    emb2 = step_embedding.reshape(step_embedding.shape[0], C)
    step1 = jnp.asarray(step, jnp.int32).reshape(1)

    row = _sc_lookup(step1, emb2)
    out = _tc_add(x2, row)
    return out.reshape(B, S, C)


# speedup vs baseline: 1.8726x; 1.0139x over previous
import functools

import jax
import jax.numpy as jnp
from jax import lax
from jax.experimental import pallas as pl
from jax.experimental.pallas import tpu as pltpu
from jax.experimental.pallas import tpu_sc as plsc

_NC = 2
_NS = 16

_C = 1024
_BLK = 2048


def _lookup_body(step_hbm, emb_hbm, out_hbm, step_smem):
    pltpu.sync_copy(step_hbm, step_smem)
    s = step_smem[0]
    pltpu.sync_copy(emb_hbm.at[s], out_hbm)


def _sc_lookup(step1, emb2):
    mesh = plsc.ScalarSubcoreMesh(axis_name="c", num_cores=1)
    run = functools.partial(
        pl.kernel,
        out_type=jax.ShapeDtypeStruct((_C,), jnp.float32),
        mesh=mesh,
        scratch_types=[
            pltpu.SMEM((1,), jnp.int32),
        ],
    )(_lookup_body)
    return run(step1, emb2).reshape(1, _C)


def _tc_body(x_ref, row_ref, out_ref):
    out_ref[...] = x_ref[...] + row_ref[...]


def _tc_add(x2, row):
    n_rows = x2.shape[0]
    return pl.pallas_call(
        _tc_body,
        grid=(n_rows // _BLK,),
        in_specs=[
            pl.BlockSpec((_BLK, _C), lambda i: (i, 0)),
            pl.BlockSpec((1, _C), lambda i: (0, 0)),
        ],
        out_specs=pl.BlockSpec((_BLK, _C), lambda i: (i, 0)),
        out_shape=jax.ShapeDtypeStruct((n_rows, _C), jnp.float32),
    )(x2, row)


def kernel(x_layer, step, step_embedding):
    B, S, C = x_layer.shape
    x2 = x_layer.reshape(B * S, C)
    emb2 = step_embedding.reshape(step_embedding.shape[0], C)
    step1 = jnp.asarray(step, jnp.int32).reshape(1)

    row = _sc_lookup(step1, emb2)
    out = _tc_add(x2, row)
    return out.reshape(B, S, C)


# speedup vs baseline: 1.8750x; 1.0013x over previous
import functools

import jax
import jax.numpy as jnp
from jax.experimental import pallas as pl
from jax.experimental.pallas import tpu as pltpu
from jax.experimental.pallas import tpu_sc as plsc

_C = 1024
_BLK = 2048


def _lookup_body(step_hbm, emb_hbm, out_hbm, step_smem):
    pltpu.sync_copy(step_hbm, step_smem)
    s = step_smem[0]
    pltpu.sync_copy(emb_hbm.at[s], out_hbm)


def _sc_lookup(step1, emb2):
    mesh = plsc.ScalarSubcoreMesh(axis_name="c", num_cores=1)
    run = functools.partial(
        pl.kernel,
        out_type=jax.ShapeDtypeStruct((_C,), jnp.float32),
        mesh=mesh,
        scratch_types=[
            pltpu.SMEM((1,), jnp.int32),
        ],
    )(_lookup_body)
    return run(step1, emb2).reshape(1, _C)


def _tc_body(x_ref, row_ref, out_ref):
    out_ref[...] = x_ref[...] + row_ref[...]


def _tc_add(x2, row):
    n_rows = x2.shape[0]
    return pl.pallas_call(
        _tc_body,
        grid=(n_rows // _BLK,),
        in_specs=[
            pl.BlockSpec((_BLK, _C), lambda i: (i, 0)),
            pl.BlockSpec((1, _C), lambda i: (0, 0)),
        ],
        out_specs=pl.BlockSpec((_BLK, _C), lambda i: (i, 0)),
        out_shape=jax.ShapeDtypeStruct((n_rows, _C), jnp.float32),
    )(x2, row)


def kernel(x_layer, step, step_embedding):
    B, S, C = x_layer.shape
    x2 = x_layer.reshape(B * S, C)
    emb2 = step_embedding.reshape(step_embedding.shape[0], C)
    step1 = jnp.asarray(step, jnp.int32).reshape(1)

    row = _sc_lookup(step1, emb2)
    out = _tc_add(x2, row)
    return out.reshape(B, S, C)
